# baseline (speedup 1.0000x reference)
import functools

import jax
import jax.numpy as jnp
from jax import lax
from jax.experimental import pallas as pl
from jax.experimental.pallas import tpu as pltpu
from jax.experimental.pallas import tpu_sc as plsc

G, N, E, D, HID, HEADS, NCLS = 4, 10000, 320000, 128, 256, 8, 10

NCORES, NSUB = 2, 16
NW = NCORES * NSUB
EW = E // NW
EWP = 10240
NPD = 10240
TN = NPD // NW
CBR = 64
RCAP = 12288
NOFF = 48
MAGIC = 26215

BT = 1024
NB = 10

_i16 = lambda v: jnp.full((16,), v, jnp.int32)


def _vgather(v, idx):
    dn = lax.GatherDimensionNumbers(offset_dims=(), collapsed_slice_dims=(0,),
                                    start_index_map=(0,))
    return lax.gather(v, idx[:, None], dn, (1,),
                      mode=lax.GatherScatterMode.PROMISE_IN_BOUNDS)


def _deg_body(dst_hbm, zeros_hbm, deg_out, hist, idx_v):
    c = lax.axis_index("c")
    s = lax.axis_index("s")
    w = c * NSUB + s
    ones = jnp.full((16,), 1.0, jnp.float32)
    for gi in range(G):
        pltpu.sync_copy(zeros_hbm, hist)
        pltpu.sync_copy(dst_hbm.at[gi, w], idx_v)

        def body(i, _):
            idx = idx_v[pl.ds(i * 16, 16)]
            plsc.addupdate_scatter(hist, [idx], ones)
            return ()

        lax.fori_loop(0, EWP // 16, body, (), unroll=4)
        for nb in range(NB):
            pltpu.sync_copy(hist.at[pl.ds(nb * BT, BT)], deg_out.at[gi, nb, w])


def _route_body(src_hbm, dst_hbm, rsrc_out, rdl_out, roff_out,
                src_v, dst_v, rsrc_v, rdl_v, hist, offsb, offw):
    c = lax.axis_index("c")
    s = lax.axis_index("s")
    w = c * NSUB + s
    iota = lax.iota(jnp.int32, 16)
    ones = _i16(1)
    zero16 = _i16(0)
    previ = jnp.maximum(iota - 1, 0)

    def graph(gi, _):
        pltpu.sync_copy(src_hbm.at[gi, w], src_v)
        pltpu.sync_copy(dst_hbm.at[gi, w], dst_v)

        def clear(i, _):
            rsrc_v[pl.ds(i * 16, 16)] = zero16
            rdl_v[pl.ds(i * 16, 16)] = _i16(TN)
            return ()

        lax.fori_loop(0, RCAP // 16, clear, (), unroll=4)
        for cchunk in range(NOFF // 16):
            hist[pl.ds(cchunk * 16, 16)] = zero16

        def p1(i, _):
            d16 = dst_v[pl.ds(i * 16, 16)]
            b = lax.shift_right_logical(d16 * MAGIC, 23)
            plsc.addupdate_scatter(hist, [b], ones)
            return ()

        lax.fori_loop(0, EWP // 16, p1, (), unroll=4)

        h0 = hist[pl.ds(0, 16)]
        h1 = hist[pl.ds(16, 16)]
        h2 = hist[pl.ds(32, 16)]
        m = jnp.int32(~63)
        r0 = (h0 + 63) & m
        r1 = (h1 + 63) & m
        r2 = (h2 + 63) & m
        c0 = plsc.cumsum(r0)
        c1 = plsc.cumsum(r1) + jnp.sum(r0)
        c2 = plsc.cumsum(r2) + jnp.sum(r0) + jnp.sum(r1)
        e0, e1, e2 = c0 - r0, c1 - r1, c2 - r2
        offsb[pl.ds(0, 16)] = e0
        offsb[pl.ds(16, 16)] = e1
        offsb[pl.ds(32, 16)] = e2
        offw[pl.ds(0, 16)] = e0
        offw[pl.ds(16, 16)] = e1
        offw[pl.ds(32, 16)] = e2
        pltpu.sync_copy(offsb, roff_out.at[gi, w])

        def p2(i, _):
            d16 = dst_v[pl.ds(i * 16, 16)]
            s16 = src_v[pl.ds(i * 16, 16)]
            b = lax.shift_right_logical(d16 * MAGIC, 23)
            dl = d16 - b * TN
            bs, ls = plsc.sort_key_val(b, iota)
            prev = _vgather(bs, previ)
            change = (bs != prev) | (iota == 0)
            strt = plsc.cummax(jnp.where(change, iota, 0))
            rank = iota - strt
            base = plsc.load_gather(offw, [bs])
            pos = base + rank
            plsc.store_scatter(rsrc_v, [pos], _vgather(s16, ls))
            plsc.store_scatter(rdl_v, [pos], _vgather(dl, ls))
            plsc.addupdate_scatter(offw, [bs], ones)
            return ()

        lax.fori_loop(0, EWP // 16, p2, (), unroll=2)
        base = pl.multiple_of((gi * NW + w) * RCAP, RCAP)
        pltpu.sync_copy(rsrc_v, rsrc_out.at[pl.ds(base, RCAP)])
        pltpu.sync_copy(rdl_v, rdl_out.at[pl.ds(base, RCAP)])
        return ()

    lax.fori_loop(0, G, graph, ())


def _scat_body(hflat_hbm, rsrc_hbm, rdl_hbm, roff_hbm, part_out,
               acc, offv, sidx, dlv, buf, gsem):
    c = lax.axis_index("c")
    s = lax.axis_index("s")
    t = c * NSUB + s
    iota = lax.iota(jnp.int32, 16)
    cols = [iota + 16 * k for k in range(8)]

    def graph(gi, _):
        pltpu.sync_copy(hflat_hbm.at[pl.ds(pl.multiple_of(gi * NPD + t * TN, TN), TN)],
                        acc.at[pl.ds(0, TN)])
        pltpu.sync_copy(roff_hbm.at[gi], offv)

        def sel(row, q):
            acc_ = jnp.int32(0)
            for ck in range(3):
                v = offv[row, pl.ds(ck * 16, 16)]
                acc_ = acc_ + jnp.sum(jnp.where(iota + ck * 16 == q, v, 0))
            return acc_

        def worker(w2, _):
            start = sel(w2, t)
            end = sel(w2, t + 1)
            n64 = lax.shift_right_logical(end - start, 6)

            base = (gi * NW + w2) * RCAP

            def chunk(ch, _):
                off = pl.multiple_of(base + start + ch * CBR, CBR)
                pltpu.sync_copy(rsrc_hbm.at[pl.ds(off, CBR)], sidx)
                pltpu.sync_copy(rdl_hbm.at[pl.ds(off, CBR)], dlv)
                pltpu.async_copy(hflat_hbm.at[sidx], buf, gsem).wait()
                dchunks = [dlv[pl.ds(cc * 16, 16)] for cc in range(4)]
                for j in range(CBR):
                    dlb = _vgather(dchunks[j // 16], _i16(j % 16))
                    for k in range(8):
                        v = buf[j, pl.ds(16 * k, 16)]
                        plsc.addupdate_scatter(acc, [dlb, cols[k]], v)
                return ()

            lax.fori_loop(0, n64, chunk, ())
            return ()

        lax.fori_loop(0, NW, worker, ())
        pltpu.sync_copy(acc.at[pl.ds(0, TN)],
                        part_out.at[gi, pl.ds(t * TN, TN)])
        return ()

    lax.fori_loop(0, G, graph, ())


@functools.cache
def _sc_kernels():
    mesh = plsc.VectorSubcoreMesh(core_axis_name="c", subcore_axis_name="s",
                                  num_cores=NCORES, num_subcores=NSUB)
    sc_params = pltpu.CompilerParams(needs_layout_passes=False)
    deg_k = pl.kernel(
        _deg_body,
        out_type=jax.ShapeDtypeStruct((G, NB, NW, BT), jnp.float32),
        mesh=mesh,
        compiler_params=sc_params,
        scratch_types=[
            pltpu.VMEM((NPD,), jnp.float32),
            pltpu.VMEM((EWP,), jnp.int32),
        ],
    )
    route_k = pl.kernel(
        _route_body,
        out_type=[
            jax.ShapeDtypeStruct((G * NW * RCAP,), jnp.int32),
            jax.ShapeDtypeStruct((G * NW * RCAP,), jnp.int32),
            jax.ShapeDtypeStruct((G, NW, NOFF), jnp.int32),
        ],
        mesh=mesh,
        compiler_params=sc_params,
        scratch_types=[
            pltpu.VMEM((EWP,), jnp.int32),
            pltpu.VMEM((EWP,), jnp.int32),
            pltpu.VMEM((RCAP,), jnp.int32),
            pltpu.VMEM((RCAP,), jnp.int32),
            pltpu.VMEM((NOFF,), jnp.int32),
            pltpu.VMEM((NOFF,), jnp.int32),
            pltpu.VMEM((NOFF,), jnp.int32),
        ],
    )
    scat_k = pl.kernel(
        _scat_body,
        out_type=jax.ShapeDtypeStruct((G, NPD, D), jnp.float32),
        mesh=mesh,
        compiler_params=sc_params,
        scratch_types=[
            pltpu.VMEM((TN + 8, D), jnp.float32),
            pltpu.VMEM((NW, NOFF), jnp.int32),
            pltpu.VMEM((CBR,), jnp.int32),
            pltpu.VMEM((CBR,), jnp.int32),
            pltpu.VMEM((CBR, D), jnp.float32),
            pltpu.SemaphoreType.DMA,
        ],
    )
    return deg_k, route_k, scat_k


def _l1_body(x_ref, degp_ref, w_ref, h_ref, dis_ref):
    degt = jnp.transpose(degp_ref[0, 0])
    deg = jnp.sum(degt, axis=1, keepdims=True) + 1.0
    dis = lax.rsqrt(deg)
    dis_ref[0] = dis
    h = jnp.dot(x_ref[0], w_ref[...], preferred_element_type=jnp.float32)
    h_ref[0] = h * dis


def _l1_call(x, degp, w1t):
    return pl.pallas_call(
        _l1_body,
        grid=(G, NB),
        in_specs=[
            pl.BlockSpec((1, BT, D), lambda g, nb: (g, nb, 0)),
            pl.BlockSpec((1, 1, NW, BT), lambda g, nb: (g, nb, 0, 0)),
            pl.BlockSpec((D, D), lambda g, nb: (0, 0)),
        ],
        out_specs=[
            pl.BlockSpec((1, BT, D), lambda g, nb: (g, nb, 0)),
            pl.BlockSpec((1, BT, 1), lambda g, nb: (g, nb, 0)),
        ],
        out_shape=[
            jax.ShapeDtypeStruct((G, NPD, D), jnp.float32),
            jax.ShapeDtypeStruct((G, NPD, 1), jnp.float32),
        ],
    )(x, degp, w1t)


def _mid_body(p_ref, dis_ref, b_ref, w_ref, out_ref):
    d = dis_ref[0]
    h = jnp.tanh(p_ref[0] * d + b_ref[0][None, :])
    out_ref[0] = jnp.dot(h, w_ref[...], preferred_element_type=jnp.float32) * d


def _mid_call(part, dis, b1r, w2t):
    return pl.pallas_call(
        _mid_body,
        grid=(G, NB),
        in_specs=[
            pl.BlockSpec((1, BT, D), lambda g, nb: (g, nb, 0)),
            pl.BlockSpec((1, BT, 1), lambda g, nb: (g, nb, 0)),
            pl.BlockSpec((1, D), lambda g, nb: (0, 0)),
            pl.BlockSpec((D, D), lambda g, nb: (0, 0)),
        ],
        out_specs=pl.BlockSpec((1, BT, D), lambda g, nb: (g, nb, 0)),
        out_shape=jax.ShapeDtypeStruct((G, NPD, D), jnp.float32),
    )(part, dis, b1r, w2t)


def _sum_body(p_ref, dis_ref, b_ref, x_ref):
    g = pl.program_id(0)
    nb = pl.program_id(1)
    d = dis_ref[0]
    h = jnp.tanh(p_ref[0] * d + b_ref[0][None, :])
    rows = lax.broadcasted_iota(jnp.int32, (BT, D), 0) + nb * BT
    h = jnp.where(rows < N, h, 0.0)
    colsum = jnp.sum(h, axis=0)

    @pl.when(nb == 0)
    def _():
        x_ref[pl.ds(g, 1), :] = colsum[None, :]

    @pl.when(nb > 0)
    def _():
        x_ref[pl.ds(g, 1), :] = x_ref[pl.ds(g, 1), :] + colsum[None, :]


def _sum_call(part, dis, b2r):
    return pl.pallas_call(
        _sum_body,
        grid=(G, NB),
        in_specs=[
            pl.BlockSpec((1, BT, D), lambda g, nb: (g, nb, 0)),
            pl.BlockSpec((1, BT, 1), lambda g, nb: (g, nb, 0)),
            pl.BlockSpec((1, D), lambda g, nb: (0, 0)),
        ],
        out_specs=pl.BlockSpec((G, D), lambda g, nb: (0, 0)),
        out_shape=jax.ShapeDtypeStruct((G, D), jnp.float32),
    )(part, dis, b2r)


def _head_body(x_ref, wq, bq, wk, bk, wv, bv, wo, bo, wm1, bm1, wm2, bm2,
               g2r, beta2r, wl, bl, logits_ref, node_ref):
    X = x_ref[...]
    q = jnp.dot(X, wq[...], preferred_element_type=jnp.float32) + bq[0][None, :]
    k = jnp.dot(X, wk[...], preferred_element_type=jnp.float32) + bk[0][None, :]
    v = jnp.dot(X, wv[...], preferred_element_type=jnp.float32) + bv[0][None, :]
    dh = D // HEADS
    outs = []
    for h in range(HEADS):
        sl = slice(h * dh, (h + 1) * dh)
        qh, kh, vh = q[:, sl], k[:, sl], v[:, sl]
        sh = lax.dot_general(qh, kh, (((1,), (1,)), ((), ())),
                             preferred_element_type=jnp.float32)
        sh = sh / (float(dh) ** 0.5)
        m = jnp.max(sh, axis=-1, keepdims=True)
        e = jnp.exp(sh - m)
        a = e / jnp.sum(e, axis=-1, keepdims=True)
        outs.append(jnp.dot(a, vh, preferred_element_type=jnp.float32))
    o = jnp.concatenate(outs, axis=1)
    x_at = jnp.dot(o, wo[...], preferred_element_type=jnp.float32) + bo[0][None, :]
    mm = jnp.maximum(
        jnp.dot(x_at, wm1[...], preferred_element_type=jnp.float32)
        + bm1[0][None, :], 0.0)
    mm = jnp.dot(mm, wm2[...], preferred_element_type=jnp.float32) + bm2[0][None, :]
    y = x_at + mm
    mu = jnp.mean(y, axis=-1, keepdims=True)
    var = jnp.mean((y - mu) ** 2, axis=-1, keepdims=True)
    y = (y - mu) / jnp.sqrt(var + 1e-5) * g2r[0][None, :] + beta2r[0][None, :]
    xr = jnp.maximum(y, 0.0)
    node = jnp.sum(xr, axis=0, keepdims=True)
    node_ref[...] = node
    logits_ref[...] = jnp.dot(node, wl[...],
                              preferred_element_type=jnp.float32) + bl[0][None, :]


def _head_call(Xg, *weights):
    return pl.pallas_call(
        _head_body,
        out_shape=[
            jax.ShapeDtypeStruct((1, NCLS), jnp.float32),
            jax.ShapeDtypeStruct((1, D), jnp.float32),
        ],
    )(Xg, *weights)


def kernel(x, edge_index, W1, b1, W2, b2, Wq, bq, Wk, bk, Wv, bv, Wo, bo,
           Wm1, bm1, Wm2, bm2, g2, beta2, Wl, bl):
    f32 = jnp.float32
    src = edge_index[:, 0, :].astype(jnp.int32)
    dst = edge_index[:, 1, :].astype(jnp.int32)
    offs = (jnp.arange(G, dtype=jnp.int32) * NPD)[:, None]
    pad = ((0, 0), (0, 0), (0, EWP - EW))
    src3 = jnp.pad((src + offs).reshape(G, NW, EW), pad, constant_values=0)
    dst3 = jnp.pad(dst.reshape(G, NW, EW), pad, constant_values=N + 16)
    zeros_np = jnp.zeros((NPD,), f32)

    r1 = lambda a: a.reshape(1, -1)
    w1t, w2t = W1.T, W2.T

    deg_k, route_k, scat_k = _sc_kernels()
    rsrc, rdl, roff = route_k(src3, dst3)
    degp = deg_k(dst3, zeros_np)
    h1p, dis = _l1_call(x, degp, w1t)

    part1 = scat_k(h1p.reshape(G * NPD, D), rsrc, rdl, roff)
    h2p = _mid_call(part1, dis, r1(b1), w2t)

    part2 = scat_k(h2p.reshape(G * NPD, D), rsrc, rdl, roff)
    Xg = _sum_call(part2, dis, r1(b2))

    logits2, node2 = _head_call(
        Xg, Wq.T, r1(bq), Wk.T, r1(bk), Wv.T, r1(bv), Wo.T, r1(bo),
        Wm1.T, r1(bm1), Wm2.T, r1(bm2), r1(g2), r1(beta2), Wl.T, r1(bl))
    return (logits2[0], node2[0])


# speedup vs baseline: 4.5454x; 4.5454x over previous
import functools

import jax
import jax.numpy as jnp
from jax import lax
from jax.experimental import pallas as pl
from jax.experimental.pallas import tpu as pltpu
from jax.experimental.pallas import tpu_sc as plsc

G, N, E, D, HID, HEADS, NCLS = 4, 10000, 320000, 128, 256, 8, 10

NCORES, NSUB = 2, 16
NW = NCORES * NSUB
EW = E // NW
CB = 64
NCH = 160
EWP = NCH * CB
NCS = NCH // 4
NPD = 10240
NROW = NPD // NSUB

BT = 1024
NB = 10

def _deg_body(dst_hbm, zeros_hbm, deg_out, hist, idx_v):
    c = lax.axis_index("c")
    s = lax.axis_index("s")
    w = c * NSUB + s
    ones = jnp.full((16,), 1.0, jnp.float32)
    for gi in range(G):
        pltpu.sync_copy(zeros_hbm, hist)
        pltpu.sync_copy(dst_hbm.at[gi, w], idx_v)

        def body(i, _):
            idx = idx_v[pl.ds(i * 16, 16)]
            plsc.addupdate_scatter(hist, [idx], ones)
            return ()

        lax.fori_loop(0, EWP // 16, body, (), unroll=4)
        for nb in range(NB):
            pltpu.sync_copy(hist.at[pl.ds(nb * BT, BT)], deg_out.at[gi, nb, w])


def _scat_body(hflat_hbm, src_hbm, dst_hbm, zeros_hbm, part_out,
               src_v, dst_v, buf0, buf1, buf2, buf3, acc,
               g0, g1, g2, g3, s0, s1, s2, s3):
    c = lax.axis_index("c")
    s = lax.axis_index("s")
    w = c * NSUB + s
    for gi in range(G):
        @pl.when(c == 0)
        def _():
            pltpu.sync_copy(hflat_hbm.at[pl.ds(gi * NPD + s * NROW, NROW)],
                            acc.at[pl.ds(s * NROW, NROW)])
        @pl.when(c != 0)
        def _():
            pltpu.sync_copy(zeros_hbm.at[pl.ds(s * NROW, NROW)],
                            acc.at[pl.ds(s * NROW, NROW)])
        plsc.subcore_barrier()

        bufs = (buf0, buf1, buf2, buf3)
        gsems = (g0, g1, g2, g3)
        ssems = (s0, s1, s2, s3)

        def step(t, _):
            for b in range(4):
                ch = 4 * t + b
                pltpu.make_async_copy(hflat_hbm.at[pl.ds(0, CB)], bufs[b],
                                      gsems[b]).wait()
                pltpu.async_copy(bufs[b], acc.at[dst_v.at[ch]], ssems[b],
                                 add=True)
            for b in range(4):
                ch = 4 * t + b
                pltpu.make_async_copy(bufs[b], acc.at[pl.ds(0, CB)],
                                      ssems[b]).wait()

                @pl.when(ch + 4 < NCS)
                def _():
                    pltpu.async_copy(hflat_hbm.at[src_v.at[ch + 4]], bufs[b],
                                     gsems[b])
            return ()

        for hc in range(NCH // NCS):
            pltpu.sync_copy(src_hbm.at[gi, w, pl.ds(hc * NCS, NCS)], src_v)
            pltpu.sync_copy(dst_hbm.at[gi, w, pl.ds(hc * NCS, NCS)], dst_v)
            for b in range(4):
                pltpu.async_copy(hflat_hbm.at[src_v.at[b]], bufs[b], gsems[b])
            lax.fori_loop(0, NCS // 4, step, ())
        plsc.subcore_barrier()
        pltpu.sync_copy(acc.at[pl.ds(s * NROW, NROW)],
                        part_out.at[c, gi, pl.ds(s * NROW, NROW)])
        plsc.subcore_barrier()


@functools.cache
def _sc_kernels():
    mesh = plsc.VectorSubcoreMesh(core_axis_name="c", subcore_axis_name="s",
                                  num_cores=NCORES, num_subcores=NSUB)
    sc_params = pltpu.CompilerParams(needs_layout_passes=False)
    deg_k = pl.kernel(
        _deg_body,
        out_type=jax.ShapeDtypeStruct((G, NB, NW, BT), jnp.float32),
        mesh=mesh,
        compiler_params=sc_params,
        scratch_types=[
            pltpu.VMEM((NPD,), jnp.float32),
            pltpu.VMEM((EWP,), jnp.int32),
        ],
    )
    scat_k = pl.kernel(
        _scat_body,
        out_type=jax.ShapeDtypeStruct((NCORES, G, NPD, D), jnp.float32),
        mesh=mesh,
        compiler_params=sc_params,
        scratch_types=[
            pltpu.VMEM((NCS, CB), jnp.int32),
            pltpu.VMEM((NCS, CB), jnp.int32),
            pltpu.VMEM((CB, D), jnp.float32),
            pltpu.VMEM((CB, D), jnp.float32),
            pltpu.VMEM((CB, D), jnp.float32),
            pltpu.VMEM((CB, D), jnp.float32),
            pltpu.MemorySpace.VMEM_SHARED((NPD, D), jnp.float32),
            pltpu.SemaphoreType.DMA,
            pltpu.SemaphoreType.DMA,
            pltpu.SemaphoreType.DMA,
            pltpu.SemaphoreType.DMA,
            pltpu.SemaphoreType.DMA,
            pltpu.SemaphoreType.DMA,
            pltpu.SemaphoreType.DMA,
            pltpu.SemaphoreType.DMA,
        ],
    )
    return deg_k, scat_k


def _l1_body(x_ref, degp_ref, w_ref, h_ref, dis_ref):
    degt = jnp.transpose(degp_ref[0, 0])
    deg = jnp.sum(degt, axis=1, keepdims=True) + 1.0
    dis = lax.rsqrt(deg)
    dis_ref[0] = dis
    h = jnp.dot(x_ref[0], w_ref[...], preferred_element_type=jnp.float32)
    h_ref[0] = h * dis


def _l1_call(x, degp, w1t):
    return pl.pallas_call(
        _l1_body,
        grid=(G, NB),
        in_specs=[
            pl.BlockSpec((1, BT, D), lambda g, nb: (g, nb, 0)),
            pl.BlockSpec((1, 1, NW, BT), lambda g, nb: (g, nb, 0, 0)),
            pl.BlockSpec((D, D), lambda g, nb: (0, 0)),
        ],
        out_specs=[
            pl.BlockSpec((1, BT, D), lambda g, nb: (g, nb, 0)),
            pl.BlockSpec((1, BT, 1), lambda g, nb: (g, nb, 0)),
        ],
        out_shape=[
            jax.ShapeDtypeStruct((G, NPD, D), jnp.float32),
            jax.ShapeDtypeStruct((G, NPD, 1), jnp.float32),
        ],
    )(x, degp, w1t)


def _mid_body(p_ref, dis_ref, b_ref, w_ref, out_ref):
    d = dis_ref[0]
    agg = p_ref[0, 0] + p_ref[1, 0]
    h = jnp.tanh(agg * d + b_ref[0][None, :])
    out_ref[0] = jnp.dot(h, w_ref[...], preferred_element_type=jnp.float32) * d


def _mid_call(part, dis, b1r, w2t):
    return pl.pallas_call(
        _mid_body,
        grid=(G, NB),
        in_specs=[
            pl.BlockSpec((2, 1, BT, D), lambda g, nb: (0, g, nb, 0)),
            pl.BlockSpec((1, BT, 1), lambda g, nb: (g, nb, 0)),
            pl.BlockSpec((1, D), lambda g, nb: (0, 0)),
            pl.BlockSpec((D, D), lambda g, nb: (0, 0)),
        ],
        out_specs=pl.BlockSpec((1, BT, D), lambda g, nb: (g, nb, 0)),
        out_shape=jax.ShapeDtypeStruct((G, NPD, D), jnp.float32),
    )(part, dis, b1r, w2t)


def _sum_body(p_ref, dis_ref, b_ref, x_ref):
    g = pl.program_id(0)
    nb = pl.program_id(1)
    d = dis_ref[0]
    agg = p_ref[0, 0] + p_ref[1, 0]
    h = jnp.tanh(agg * d + b_ref[0][None, :])
    rows = lax.broadcasted_iota(jnp.int32, (BT, D), 0) + nb * BT
    h = jnp.where(rows < N, h, 0.0)
    colsum = jnp.sum(h, axis=0)

    @pl.when(nb == 0)
    def _():
        x_ref[pl.ds(g, 1), :] = colsum[None, :]

    @pl.when(nb > 0)
    def _():
        x_ref[pl.ds(g, 1), :] = x_ref[pl.ds(g, 1), :] + colsum[None, :]


def _sum_call(part, dis, b2r):
    return pl.pallas_call(
        _sum_body,
        grid=(G, NB),
        in_specs=[
            pl.BlockSpec((2, 1, BT, D), lambda g, nb: (0, g, nb, 0)),
            pl.BlockSpec((1, BT, 1), lambda g, nb: (g, nb, 0)),
            pl.BlockSpec((1, D), lambda g, nb: (0, 0)),
        ],
        out_specs=pl.BlockSpec((G, D), lambda g, nb: (0, 0)),
        out_shape=jax.ShapeDtypeStruct((G, D), jnp.float32),
    )(part, dis, b2r)


def _head_body(x_ref, wq, bq, wk, bk, wv, bv, wo, bo, wm1, bm1, wm2, bm2,
               g2r, beta2r, wl, bl, logits_ref, node_ref):
    X = x_ref[...]
    q = jnp.dot(X, wq[...], preferred_element_type=jnp.float32) + bq[0][None, :]
    k = jnp.dot(X, wk[...], preferred_element_type=jnp.float32) + bk[0][None, :]
    v = jnp.dot(X, wv[...], preferred_element_type=jnp.float32) + bv[0][None, :]
    dh = D // HEADS
    outs = []
    for h in range(HEADS):
        sl = slice(h * dh, (h + 1) * dh)
        qh, kh, vh = q[:, sl], k[:, sl], v[:, sl]
        sh = lax.dot_general(qh, kh, (((1,), (1,)), ((), ())),
                             preferred_element_type=jnp.float32)
        sh = sh / (float(dh) ** 0.5)
        m = jnp.max(sh, axis=-1, keepdims=True)
        e = jnp.exp(sh - m)
        a = e / jnp.sum(e, axis=-1, keepdims=True)
        outs.append(jnp.dot(a, vh, preferred_element_type=jnp.float32))
    o = jnp.concatenate(outs, axis=1)
    x_at = jnp.dot(o, wo[...], preferred_element_type=jnp.float32) + bo[0][None, :]
    mm = jnp.maximum(
        jnp.dot(x_at, wm1[...], preferred_element_type=jnp.float32)
        + bm1[0][None, :], 0.0)
    mm = jnp.dot(mm, wm2[...], preferred_element_type=jnp.float32) + bm2[0][None, :]
    y = x_at + mm
    mu = jnp.mean(y, axis=-1, keepdims=True)
    var = jnp.mean((y - mu) ** 2, axis=-1, keepdims=True)
    y = (y - mu) / jnp.sqrt(var + 1e-5) * g2r[0][None, :] + beta2r[0][None, :]
    xr = jnp.maximum(y, 0.0)
    node = jnp.sum(xr, axis=0, keepdims=True)
    node_ref[...] = node
    logits_ref[...] = jnp.dot(node, wl[...],
                              preferred_element_type=jnp.float32) + bl[0][None, :]


def _head_call(Xg, *weights):
    return pl.pallas_call(
        _head_body,
        out_shape=[
            jax.ShapeDtypeStruct((1, NCLS), jnp.float32),
            jax.ShapeDtypeStruct((1, D), jnp.float32),
        ],
    )(Xg, *weights)


def kernel(x, edge_index, W1, b1, W2, b2, Wq, bq, Wk, bk, Wv, bv, Wo, bo,
           Wm1, bm1, Wm2, bm2, g2, beta2, Wl, bl):
    f32 = jnp.float32
    src = edge_index[:, 0, :].astype(jnp.int32)
    dst = edge_index[:, 1, :].astype(jnp.int32)
    offs = (jnp.arange(G, dtype=jnp.int32) * NPD)[:, None]
    srcw = (src + offs).reshape(G, NW, EW)
    dstw = dst.reshape(G, NW, EW)
    pad = ((0, 0), (0, 0), (0, EWP - EW))
    srcp = jnp.pad(srcw, pad, constant_values=0).reshape(G, NW, NCH, CB)
    dstp = jnp.pad(dstw, pad, constant_values=N).reshape(G, NW, NCH, CB)
    dst3 = dstp.reshape(G, NW, EWP)
    zeros_np = jnp.zeros((NPD,), f32)
    zeros_nd = jnp.zeros((NPD, D), f32)

    r1 = lambda a: a.reshape(1, -1)
    w1t, w2t = W1.T, W2.T

    deg_k, scat_k = _sc_kernels()
    degp = deg_k(dst3, zeros_np)
    h1p, dis = _l1_call(x, degp, w1t)

    part1 = scat_k(h1p.reshape(G * NPD, D), srcp, dstp, zeros_nd)
    h2p = _mid_call(part1, dis, r1(b1), w2t)

    part2 = scat_k(h2p.reshape(G * NPD, D), srcp, dstp, zeros_nd)
    Xg = _sum_call(part2, dis, r1(b2))

    logits2, node2 = _head_call(
        Xg, Wq.T, r1(bq), Wk.T, r1(bk), Wv.T, r1(bv), Wo.T, r1(bo),
        Wm1.T, r1(bm1), Wm2.T, r1(bm2), r1(g2), r1(beta2), Wl.T, r1(bl))
    return (logits2[0], node2[0])
